# baseline (speedup 1.0000x reference)
import functools

import jax
import jax.numpy as jnp
from jax import lax
from jax.experimental import pallas as pl
from jax.experimental.pallas import tpu as pltpu
from jax.experimental.pallas import tpu_sc as plsc

_P = 8
_UNROLL = 6


def kernel(x, pe_weight):
    B, S, D = x.shape
    mesh = plsc.VectorSubcoreMesh(core_axis_name="c", subcore_axis_name="s")
    nw = mesh.num_cores * mesh.num_subcores
    rows_per_w = S // nw
    nchunks = rows_per_w // _P
    dchunks = D // 16

    scratch = (
        [pltpu.VMEM((B, _P, D), jnp.float32) for _ in range(2)]
        + [pltpu.VMEM((B, _P, D), jnp.float32) for _ in range(2)]
        + [pltpu.VMEM((_P, D), jnp.float32) for _ in range(2)]
        + [pltpu.SemaphoreType.DMA for _ in range(2)]
        + [pltpu.SemaphoreType.DMA for _ in range(2 * B)]
        + [pltpu.SemaphoreType.DMA for _ in range(2)]
    )

    @functools.partial(
        pl.kernel,
        out_type=jax.ShapeDtypeStruct((B, S, D), jnp.float32),
        mesh=mesh,
        scratch_types=scratch,
    )
    def run(x_hbm, pe_hbm, out_hbm, *bufs):
        xin = bufs[0:2]
        xout = bufs[2:4]
        pebuf = bufs[4:6]
        in_sem = bufs[6:8]
        out_sem = bufs[8 : 8 + 2 * B]
        pe_sem = bufs[8 + 2 * B :]

        wid = lax.axis_index("s") * mesh.num_cores + lax.axis_index("c")
        base = wid * rows_per_w

        def seq0(c):
            return base + c * _P

        def pe_copy(c, par):
            return pltpu.make_async_copy(
                pe_hbm.at[pl.ds(seq0(c), _P)], pebuf[par], pe_sem[par]
            )

        def in_copy(c, par):
            return pltpu.make_async_copy(
                x_hbm.at[:, pl.ds(seq0(c), _P)], xin[par], in_sem[par]
            )

        def out_copy(c, par, b):
            return pltpu.make_async_copy(
                xout[par].at[b],
                out_hbm.at[b, pl.ds(seq0(c), _P)],
                out_sem[par * B + b],
            )

        for par in range(2):
            pe_copy(par, par).start()
            in_copy(par, par).start()

        def chunk_pair(c0, _):
            for par in range(2):
                c = 2 * c0 + par
                pe_copy(c, par).wait()
                in_copy(c, par).wait()
                def _drain(c=c, par=par):
                    for b in range(B):
                        out_copy(c - 2, par, b).wait()

                pl.when(c >= 2)(_drain)
                pe_v = pebuf[par]
                xi = xin[par]
                xo = xout[par]
                for b in range(B):

                    def row_body(i, _):
                        def col_body(j, _):
                            for u in range(_UNROLL):
                                off = (j * _UNROLL + u) * 16
                                xo[b, i, pl.ds(off, 16)] = (
                                    xi[b, i, pl.ds(off, 16)]
                                    + pe_v[i, pl.ds(off, 16)]
                                )
                            return 0

                        return lax.fori_loop(0, dchunks // _UNROLL, col_body, 0)

                    lax.fori_loop(0, _P, row_body, 0)
                    out_copy(c, par, b).start()
                def _prefetch(c=c, par=par):
                    in_copy(c + 2, par).start()
                    pe_copy(c + 2, par).start()

                pl.when(c + 2 < nchunks)(_prefetch)
            return 0

        lax.fori_loop(0, nchunks // 2, chunk_pair, 0)

        for par in range(2):
            for b in range(B):
                out_copy(nchunks - 2 + par, par, b).wait()

    return run(x, pe_weight)


# speedup vs baseline: 1.0680x; 1.0680x over previous
import functools

import jax
import jax.numpy as jnp
from jax import lax
from jax.experimental import pallas as pl
from jax.experimental.pallas import tpu as pltpu
from jax.experimental.pallas import tpu_sc as plsc

_P = 8
_UNROLL = 8


def kernel(x, pe_weight):
    B, S, D = x.shape
    mesh = plsc.VectorSubcoreMesh(core_axis_name="c", subcore_axis_name="s")
    nw = mesh.num_cores * mesh.num_subcores
    rows_per_w = S // nw
    nchunks = rows_per_w // _P
    F = _P * D

    scratch = (
        [pltpu.VMEM((B, F), jnp.float32) for _ in range(2)]
        + [pltpu.VMEM((B, F), jnp.float32) for _ in range(2)]
        + [pltpu.VMEM((F,), jnp.float32) for _ in range(2)]
        + [pltpu.SemaphoreType.DMA for _ in range(2)]
        + [pltpu.SemaphoreType.DMA for _ in range(2 * B)]
        + [pltpu.SemaphoreType.DMA for _ in range(2)]
    )

    @functools.partial(
        pl.kernel,
        out_type=jax.ShapeDtypeStruct((B, S * D), jnp.float32),
        mesh=mesh,
        scratch_types=scratch,
    )
    def run(x_hbm, pe_hbm, out_hbm, *bufs):
        xin = bufs[0:2]
        xout = bufs[2:4]
        pebuf = bufs[4:6]
        in_sem = bufs[6:8]
        out_sem = bufs[8 : 8 + 2 * B]
        pe_sem = bufs[8 + 2 * B :]

        wid = lax.axis_index("s") * mesh.num_cores + lax.axis_index("c")
        base = wid * rows_per_w

        def off0(c):
            return (base + c * _P) * D

        def pe_copy(c, par):
            return pltpu.make_async_copy(
                pe_hbm.at[pl.ds(off0(c), F)], pebuf[par], pe_sem[par]
            )

        def in_copy(c, par):
            return pltpu.make_async_copy(
                x_hbm.at[:, pl.ds(off0(c), F)], xin[par], in_sem[par]
            )

        def out_copy(c, par, b):
            return pltpu.make_async_copy(
                xout[par].at[b],
                out_hbm.at[b, pl.ds(off0(c), F)],
                out_sem[par * B + b],
            )

        for par in range(2):
            pe_copy(par, par).start()
            in_copy(par, par).start()

        def chunk_pair(c0, _):
            for par in range(2):
                c = 2 * c0 + par
                pe_copy(c, par).wait()
                in_copy(c, par).wait()

                def _drain(c=c, par=par):
                    for b in range(B):
                        out_copy(c - 2, par, b).wait()

                pl.when(c >= 2)(_drain)
                pe_v = pebuf[par]
                xi = xin[par]
                xo = xout[par]
                for b in range(B):

                    @plsc.parallel_loop(0, F, 16, unroll=_UNROLL)
                    def _add(off, b=b, xi=xi, xo=xo, pe_v=pe_v):
                        xo[b, pl.ds(off, 16)] = (
                            xi[b, pl.ds(off, 16)] + pe_v[pl.ds(off, 16)]
                        )

                    out_copy(c, par, b).start()

                def _prefetch(c=c, par=par):
                    in_copy(c + 2, par).start()
                    pe_copy(c + 2, par).start()

                pl.when(c + 2 < nchunks)(_prefetch)
            return 0

        lax.fori_loop(0, nchunks // 2, chunk_pair, 0)

        for par in range(2):
            for b in range(B):
                out_copy(nchunks - 2 + par, par, b).wait()

    return run(x.reshape(B, S * D), pe_weight.reshape(S * D)).reshape(B, S, D)
